# Initial kernel scaffold; baseline (speedup 1.0000x reference)
import jax
import jax.numpy as jnp
from jax.experimental import pallas as pl


def kernel(x, edge_index, graph_ids, pgNodeEmb, neighEmb, classWeight, W_init, b_init, g1, be1, g2, be2, Wf, bf, gb, bb, Wf2, bf2, gb2, bb2, Wf3, bf3, gb3, bb3, Wf4, bf4):
    raise NotImplementedError("write your pallas kernel here")


# speedup vs baseline: 2.9189x; 2.9189x over previous
import functools

import jax
import jax.numpy as jnp
from jax import lax
from jax.experimental import pallas as pl
from jax.experimental.pallas import tpu as pltpu
from jax.experimental.pallas import tpu_sc as plsc

_N = 10000
_E = 160000
_B = 64
_K = 10
_H = 256
_HALF = 128
_NT = 16
_CH = 128
_NCH = 80
_EPAD = _NT * _NCH * _CH
_AR = 10240
_RPT = _AR // _NT
_TRASH = _N + 100


@functools.cache
def _make_agg():
    scratch = [
        pltpu.VMEM((_NCH, _CH), jnp.int32),
        pltpu.VMEM((_NCH, _CH), jnp.int32),
        pltpu.VMEM((_CH, _HALF), jnp.float32),
        pltpu.VMEM_SHARED((_AR, _HALF), jnp.float32),
        pltpu.SemaphoreType.DMA,
    ]
    mesh = plsc.VectorSubcoreMesh(core_axis_name="c", subcore_axis_name="s")

    @functools.partial(
        pl.kernel,
        out_type=jax.ShapeDtypeStruct((2, _AR, _HALF), jnp.float32),
        mesh=mesh, scratch_types=scratch)
    def agg(h2, srci, dsti, zrows, out, src_v, dst_v, rows_v, acc_sh, sem):
        cid = lax.axis_index("c")
        sid = lax.axis_index("s")
        r0 = sid * _RPT
        pltpu.sync_copy(srci.at[sid], src_v)
        pltpu.sync_copy(dsti.at[sid], dst_v)
        pltpu.sync_copy(zrows, rows_v)
        for j in range(_RPT // _CH):
            pltpu.sync_copy(rows_v, acc_sh.at[pl.ds(r0 + j * _CH, _CH)])
        plsc.subcore_barrier()

        def body(i, carry):
            pltpu.async_copy(h2.at[cid].at[src_v.at[i]], rows_v,
                             sem).wait()
            pltpu.sync_copy(rows_v, acc_sh.at[dst_v.at[i]], add=True)
            return carry

        lax.fori_loop(0, _NCH, body, 0)
        plsc.subcore_barrier()
        for j in range(_RPT // _CH):
            pltpu.sync_copy(acc_sh.at[pl.ds(r0 + j * _CH, _CH)], rows_v)
            pltpu.sync_copy(rows_v, out.at[cid, pl.ds(r0 + j * _CH, _CH)])

    return agg


@functools.cache
def _make_cnt():
    scratch = [
        pltpu.VMEM((_NCH, _CH), jnp.int32),
        pltpu.VMEM((_CH, _HALF), jnp.float32),
        pltpu.VMEM_SHARED((_AR, _HALF), jnp.float32),
    ]
    mesh = plsc.VectorSubcoreMesh(core_axis_name="c", subcore_axis_name="s")

    @functools.partial(
        pl.kernel,
        out_type=jax.ShapeDtypeStruct((2, _AR, _HALF), jnp.float32),
        mesh=mesh, scratch_types=scratch)
    def cntk(dsti, zrows, ones, cnt_out, dst_v, buf_v, cnt_sh):
        cid = lax.axis_index("c")
        sid = lax.axis_index("s")
        r0 = sid * _RPT
        half = _NCH // 2
        pltpu.sync_copy(dsti.at[sid], dst_v)
        pltpu.sync_copy(zrows, buf_v)
        for j in range(_RPT // _CH):
            pltpu.sync_copy(buf_v, cnt_sh.at[pl.ds(r0 + j * _CH, _CH)])
        pltpu.sync_copy(ones, buf_v)
        plsc.subcore_barrier()

        def body(i, carry):
            pltpu.sync_copy(buf_v, cnt_sh.at[dst_v.at[cid * half + i]],
                            add=True)
            return carry

        lax.fori_loop(0, half, body, 0)
        plsc.subcore_barrier()
        for j in range(_RPT // _CH):
            pltpu.sync_copy(cnt_sh.at[pl.ds(r0 + j * _CH, _CH)], buf_v)
            pltpu.sync_copy(buf_v, cnt_out.at[cid, pl.ds(r0 + j * _CH,
                                                         _CH)])

    return cntk


def _agg(h2, srcp, dstp):
    zrows = jnp.zeros((_CH, _HALF), jnp.float32)
    return _make_agg()(h2, srcp, dstp, zrows)[:, :_N]


def _cnt(dstp):
    zrows = jnp.zeros((_CH, _HALF), jnp.float32)
    ones = jnp.ones((_CH, _HALF), jnp.float32)
    c2 = _make_cnt()(dstp, zrows, ones)
    return c2[:, :_N, :1]


def _pad_edges(edge_index):
    pad = _EPAD - _E
    srcp = jnp.concatenate([edge_index[0], jnp.zeros((pad,), jnp.int32)]
                           ).reshape(_NT, _NCH, _CH)
    dstp = jnp.concatenate([edge_index[1],
                            jnp.full((pad,), _TRASH, jnp.int32)]
                           ).reshape(_NT, _NCH, _CH)
    return srcp, dstp


def _bn_relu(t, g, b):
    m = jnp.mean(t, axis=0, keepdims=True)
    v = jnp.mean((t - m) ** 2, axis=0, keepdims=True)
    return jnp.maximum((t - m) * lax.rsqrt(v + 1e-5) * g + b, 0.0)


def _init_proj(x, w_split, b_split):
    def body(x_ref, w_ref, b_ref, out_ref):
        out_ref[0] = (jnp.dot(x_ref[...], w_ref[0],
                              preferred_element_type=jnp.float32) + b_ref[0])

    return pl.pallas_call(
        body,
        grid=(2,),
        in_specs=[
            pl.BlockSpec((_N, 20), lambda i: (0, 0)),
            pl.BlockSpec((1, 20, _HALF), lambda i: (i, 0, 0)),
            pl.BlockSpec((1, 1, _HALF), lambda i: (i, 0, 0)),
        ],
        out_specs=pl.BlockSpec((1, _N, _HALF), lambda i: (i, 0, 0)),
        out_shape=jax.ShapeDtypeStruct((2, _N, _HALF), jnp.float32),
    )(x, w_split, b_split)


def _fold_bn_relu(h_split, s_split, cnt, g_split, be_split):
    def body(h_ref, s_ref, c_ref, g_ref, be_ref, out_ref):
        c = c_ref[0] + c_ref[1]
        inv = 1.0 / jnp.maximum(c, 1.0)
        t = h_ref[0] + s_ref[0] * inv
        out_ref[0] = _bn_relu(t, g_ref[0], be_ref[0])

    return pl.pallas_call(
        body,
        grid=(2,),
        in_specs=[
            pl.BlockSpec((1, _N, _HALF), lambda i: (i, 0, 0)),
            pl.BlockSpec((1, _N, _HALF), lambda i: (i, 0, 0)),
            pl.BlockSpec((2, _N, 1), lambda i: (0, 0, 0)),
            pl.BlockSpec((1, 1, _HALF), lambda i: (i, 0, 0)),
            pl.BlockSpec((1, 1, _HALF), lambda i: (i, 0, 0)),
        ],
        out_specs=pl.BlockSpec((1, _N, _HALF), lambda i: (i, 0, 0)),
        out_shape=jax.ShapeDtypeStruct((2, _N, _HALF), jnp.float32),
    )(h_split, s_split, cnt, g_split, be_split)


def _fold_bn_relu_pool(h_split, s_split, cnt, g_split, be_split, gids):
    def body(h_ref, s_ref, c_ref, g_ref, be_ref, id_ref, out_ref):
        c = c_ref[0] + c_ref[1]
        inv = 1.0 / jnp.maximum(c, 1.0)
        t = h_ref[0] + s_ref[0] * inv
        h2 = _bn_relu(t, g_ref[0], be_ref[0])
        onehot = (id_ref[...] ==
                  lax.broadcasted_iota(jnp.int32, (_N, _B), 1)
                  ).astype(jnp.float32)
        gsum = lax.dot_general(onehot, h2, (((0,), (0,)), ((), ())),
                               preferred_element_type=jnp.float32)
        gcnt = lax.dot_general(onehot, jnp.ones((_N, 1), jnp.float32),
                               (((0,), (0,)), ((), ())),
                               preferred_element_type=jnp.float32)
        out_ref[0] = gsum / jnp.maximum(gcnt, 1.0)

    return pl.pallas_call(
        body,
        grid=(2,),
        in_specs=[
            pl.BlockSpec((1, _N, _HALF), lambda i: (i, 0, 0)),
            pl.BlockSpec((1, _N, _HALF), lambda i: (i, 0, 0)),
            pl.BlockSpec((2, _N, 1), lambda i: (0, 0, 0)),
            pl.BlockSpec((1, 1, _HALF), lambda i: (i, 0, 0)),
            pl.BlockSpec((1, 1, _HALF), lambda i: (i, 0, 0)),
            pl.BlockSpec((_N, 1), lambda i: (0, 0)),
        ],
        out_specs=pl.BlockSpec((1, _B, _HALF), lambda i: (i, 0, 0)),
        out_shape=jax.ShapeDtypeStruct((2, _B, _HALF), jnp.float32),
    )(h_split, s_split, cnt, g_split, be_split, gids)


def _mlp_head(qemb, pg, ne, wf_q, wf_p, wf_n, bf, gb, bb,
              w2, b2, gb2, bb2, w3, b3, gb3, bb3, w4, b4):
    def body(q_ref, pg_ref, ne_ref, wfq_ref, wfp_ref, wfn_ref, bf_ref,
             gb_ref, bb_ref, w2_ref, b2_ref, gb2_ref, bb2_ref,
             w3_ref, b3_ref, gb3_ref, bb3_ref, w4_ref, b4_ref,
             preds_ref, h3_ref):
        t1 = (jnp.dot(q_ref[...], wfq_ref[...],
                      preferred_element_type=jnp.float32) +
              jnp.dot(pg_ref[...], wfp_ref[...],
                      preferred_element_type=jnp.float32))
        rep = (lax.broadcasted_iota(jnp.int32, (_B * _K, _B), 0) // _K ==
               lax.broadcasted_iota(jnp.int32, (_B * _K, _B), 1)
               ).astype(jnp.float32)
        t1r = jnp.dot(rep, t1, preferred_element_type=jnp.float32)
        t2 = jnp.dot(ne_ref[...], wfn_ref[...],
                     preferred_element_type=jnp.float32)
        h = _bn_relu(t1r + t2 + bf_ref[...], gb_ref[...], bb_ref[...])
        h2 = _bn_relu(jnp.dot(h, w2_ref[...],
                              preferred_element_type=jnp.float32)
                      + b2_ref[...], gb2_ref[...], bb2_ref[...])
        h3 = _bn_relu(jnp.dot(h2, w3_ref[...],
                              preferred_element_type=jnp.float32)
                      + b3_ref[...], gb3_ref[...], bb3_ref[...])
        h3_ref[...] = h3
        logits = jnp.dot(h3, w4_ref[...],
                         preferred_element_type=jnp.float32) + b4_ref[...]
        preds_ref[...] = 1.0 / (1.0 + jnp.exp(-logits))

    return pl.pallas_call(
        body,
        out_shape=[
            jax.ShapeDtypeStruct((_B * _K, 1), jnp.float32),
            jax.ShapeDtypeStruct((_B * _K, _H), jnp.float32),
        ],
    )(qemb, pg, ne, wf_q, wf_p, wf_n, bf, gb, bb,
      w2, b2, gb2, bb2, w3, b3, gb3, bb3, w4, b4)


def kernel(x, edge_index, graph_ids, pgNodeEmb, neighEmb, classWeight,
           W_init, b_init, g1, be1, g2, be2, Wf, bf, gb, bb,
           Wf2, bf2, gb2, bb2, Wf3, bf3, gb3, bb3, Wf4, bf4):
    srcp, dstp = _pad_edges(edge_index)
    gids = graph_ids.reshape(_N, 1)

    w_split = W_init.reshape(20, 2, _HALF).transpose(1, 0, 2)
    b_split = b_init.reshape(2, 1, _HALF)
    g1s = g1.reshape(2, 1, _HALF)
    be1s = be1.reshape(2, 1, _HALF)
    g2s = g2.reshape(2, 1, _HALF)
    be2s = be2.reshape(2, 1, _HALF)

    h0 = _init_proj(x, w_split, b_split)
    cnt = _cnt(dstp)
    s1 = _agg(h0, srcp, dstp)
    h1 = _fold_bn_relu(h0, s1, cnt, g1s, be1s)
    s2 = _agg(h1, srcp, dstp)
    q2 = _fold_bn_relu_pool(h1, s2, cnt, g2s, be2s, gids)
    qemb = jnp.concatenate([q2[0], q2[1]], axis=1)

    preds640, h3 = _mlp_head(
        qemb, pgNodeEmb, neighEmb,
        Wf[:_H], Wf[_H:2 * _H], Wf[2 * _H:],
        bf.reshape(1, _H), gb.reshape(1, _H), bb.reshape(1, _H),
        Wf2, bf2.reshape(1, _H), gb2.reshape(1, _H), bb2.reshape(1, _H),
        Wf3, bf3.reshape(1, _H), gb3.reshape(1, _H), bb3.reshape(1, _H),
        Wf4, bf4.reshape(1, 1))
    return (preds640.reshape(_B, _K), h3)


# speedup vs baseline: 3.4698x; 1.1887x over previous
import functools

import jax
import jax.numpy as jnp
from jax import lax
from jax.experimental import pallas as pl
from jax.experimental.pallas import tpu as pltpu
from jax.experimental.pallas import tpu_sc as plsc

_N = 10000
_E = 160000
_B = 64
_K = 10
_H = 256
_HALF = 128
_NT = 16
_CH = 128
_NCH = 80
_EPAD = _NT * _NCH * _CH
_AR = 10240
_RPT = _AR // _NT
_TRASH = _N + 100


_BLK = 10
_NBLK = _NCH // _BLK


@functools.cache
def _make_agg():
    scratch = [
        pltpu.VMEM((_BLK, 1, _CH), jnp.int32),
        pltpu.VMEM((_BLK, 1, _CH), jnp.int32),
        pltpu.VMEM((_BLK, 1, _CH), jnp.int32),
        pltpu.VMEM((_BLK, 1, _CH), jnp.int32),
        pltpu.VMEM((_CH, _HALF), jnp.float32),
        pltpu.VMEM((_CH, _HALF), jnp.float32),
        pltpu.VMEM_SHARED((_AR, _HALF), jnp.float32),
        pltpu.SemaphoreType.DMA,
        pltpu.SemaphoreType.DMA,
    ]
    mesh = plsc.VectorSubcoreMesh(core_axis_name="c", subcore_axis_name="s")

    @functools.partial(
        pl.kernel,
        out_type=jax.ShapeDtypeStruct((2, _AR, _HALF), jnp.float32),
        mesh=mesh, scratch_types=scratch)
    def agg(h2, srcb, dstb, zrows, out, src_a, src_b, dst_a, dst_b,
            rows0, rows1, acc_sh, sem0, sem1):
        cid = lax.axis_index("c")
        sid = lax.axis_index("s")
        r0 = sid * _RPT
        srcs = (src_a, src_b)
        dsts = (dst_a, dst_b)
        rows = (rows0, rows1)
        sems = (sem0, sem1)

        pltpu.sync_copy(zrows, rows0)
        for j in range(_RPT // _CH):
            pltpu.sync_copy(rows0, acc_sh.at[pl.ds(r0 + j * _CH, _CH)])
        pltpu.sync_copy(srcb.at[0, sid], src_a)
        pltpu.sync_copy(dstb.at[0, sid], dst_a)
        plsc.subcore_barrier()

        def start_gather(i):
            blk, j = divmod(i, _BLK)
            return pltpu.async_copy(
                h2.at[cid].at[srcs[blk % 2].at[j, 0]],
                rows[i % 2], sems[i % 2])

        pending = start_gather(0)
        for i in range(_NCH):
            blk, j = divmod(i, _BLK)
            if j == 0 and blk + 1 < _NBLK:
                pltpu.sync_copy(srcb.at[blk + 1, sid],
                                srcs[(blk + 1) % 2])
                pltpu.sync_copy(dstb.at[blk + 1, sid],
                                dsts[(blk + 1) % 2])
            nxt = start_gather(i + 1) if i + 1 < _NCH else None
            pending.wait()
            pltpu.sync_copy(rows[i % 2],
                            acc_sh.at[dsts[blk % 2].at[j, 0]], add=True)
            pending = nxt
        plsc.subcore_barrier()
        for j in range(_RPT // _CH):
            pltpu.sync_copy(acc_sh.at[pl.ds(r0 + j * _CH, _CH)], rows0)
            pltpu.sync_copy(rows0, out.at[cid, pl.ds(r0 + j * _CH, _CH)])

    return agg


@functools.cache
def _make_cnt():
    scratch = [
        pltpu.VMEM((_NCH, _CH), jnp.int32),
        pltpu.VMEM((_CH, _HALF), jnp.float32),
        pltpu.VMEM_SHARED((_AR, _HALF), jnp.float32),
    ]
    mesh = plsc.VectorSubcoreMesh(core_axis_name="c", subcore_axis_name="s")

    @functools.partial(
        pl.kernel,
        out_type=jax.ShapeDtypeStruct((2, _AR, _HALF), jnp.float32),
        mesh=mesh, scratch_types=scratch)
    def cntk(dsti, zrows, ones, cnt_out, dst_v, buf_v, cnt_sh):
        cid = lax.axis_index("c")
        sid = lax.axis_index("s")
        r0 = sid * _RPT
        half = _NCH // 2
        pltpu.sync_copy(dsti.at[sid], dst_v)
        pltpu.sync_copy(zrows, buf_v)
        for j in range(_RPT // _CH):
            pltpu.sync_copy(buf_v, cnt_sh.at[pl.ds(r0 + j * _CH, _CH)])
        pltpu.sync_copy(ones, buf_v)
        plsc.subcore_barrier()

        def body(i, carry):
            pltpu.sync_copy(buf_v, cnt_sh.at[dst_v.at[cid * half + i]],
                            add=True)
            return carry

        lax.fori_loop(0, half, body, 0)
        plsc.subcore_barrier()
        for j in range(_RPT // _CH):
            pltpu.sync_copy(cnt_sh.at[pl.ds(r0 + j * _CH, _CH)], buf_v)
            pltpu.sync_copy(buf_v, cnt_out.at[cid, pl.ds(r0 + j * _CH,
                                                         _CH)])

    return cntk


def _agg(h2, srcp, dstp):
    zrows = jnp.zeros((_CH, _HALF), jnp.float32)
    return _make_agg()(h2, srcp, dstp, zrows)[:, :_N]


def _cnt(dstp):
    zrows = jnp.zeros((_CH, _HALF), jnp.float32)
    ones = jnp.ones((_CH, _HALF), jnp.float32)
    c2 = _make_cnt()(dstp, zrows, ones)
    return c2[:, :_N, :1]


def _blk_layout(flat):
    return (flat.reshape(_NT, _NBLK, _BLK, _CH).transpose(1, 0, 2, 3)
            .reshape(_NBLK, _NT, _BLK, 1, _CH))


def _pad_edges(edge_index):
    pad = _EPAD - _E
    src_flat = jnp.concatenate([edge_index[0],
                                jnp.zeros((pad,), jnp.int32)])
    dst_flat = jnp.concatenate([edge_index[1],
                                jnp.full((pad,), _TRASH, jnp.int32)])
    return (_blk_layout(src_flat), _blk_layout(dst_flat),
            dst_flat.reshape(_NT, _NCH, _CH))


def _bn_relu(t, g, b):
    m = jnp.mean(t, axis=0, keepdims=True)
    v = jnp.mean((t - m) ** 2, axis=0, keepdims=True)
    return jnp.maximum((t - m) * lax.rsqrt(v + 1e-5) * g + b, 0.0)


def _init_proj(x, w_split, b_split):
    def body(x_ref, w_ref, b_ref, out_ref):
        out_ref[0] = (jnp.dot(x_ref[...], w_ref[0],
                              preferred_element_type=jnp.float32) + b_ref[0])

    return pl.pallas_call(
        body,
        grid=(2,),
        in_specs=[
            pl.BlockSpec((_N, 20), lambda i: (0, 0)),
            pl.BlockSpec((1, 20, _HALF), lambda i: (i, 0, 0)),
            pl.BlockSpec((1, 1, _HALF), lambda i: (i, 0, 0)),
        ],
        out_specs=pl.BlockSpec((1, _N, _HALF), lambda i: (i, 0, 0)),
        out_shape=jax.ShapeDtypeStruct((2, _N, _HALF), jnp.float32),
    )(x, w_split, b_split)


def _fold_bn_relu(h_split, s_split, cnt, g_split, be_split):
    def body(h_ref, s_ref, c_ref, g_ref, be_ref, out_ref):
        c = c_ref[0] + c_ref[1]
        inv = 1.0 / jnp.maximum(c, 1.0)
        t = h_ref[0] + s_ref[0] * inv
        out_ref[0] = _bn_relu(t, g_ref[0], be_ref[0])

    return pl.pallas_call(
        body,
        grid=(2,),
        in_specs=[
            pl.BlockSpec((1, _N, _HALF), lambda i: (i, 0, 0)),
            pl.BlockSpec((1, _N, _HALF), lambda i: (i, 0, 0)),
            pl.BlockSpec((2, _N, 1), lambda i: (0, 0, 0)),
            pl.BlockSpec((1, 1, _HALF), lambda i: (i, 0, 0)),
            pl.BlockSpec((1, 1, _HALF), lambda i: (i, 0, 0)),
        ],
        out_specs=pl.BlockSpec((1, _N, _HALF), lambda i: (i, 0, 0)),
        out_shape=jax.ShapeDtypeStruct((2, _N, _HALF), jnp.float32),
    )(h_split, s_split, cnt, g_split, be_split)


def _fold_bn_relu_pool(h_split, s_split, cnt, g_split, be_split, gids):
    def body(h_ref, s_ref, c_ref, g_ref, be_ref, id_ref, out_ref):
        c = c_ref[0] + c_ref[1]
        inv = 1.0 / jnp.maximum(c, 1.0)
        t = h_ref[0] + s_ref[0] * inv
        h2 = _bn_relu(t, g_ref[0], be_ref[0])
        onehot = (id_ref[...] ==
                  lax.broadcasted_iota(jnp.int32, (_N, _B), 1)
                  ).astype(jnp.float32)
        gsum = lax.dot_general(onehot, h2, (((0,), (0,)), ((), ())),
                               preferred_element_type=jnp.float32)
        gcnt = lax.dot_general(onehot, jnp.ones((_N, 1), jnp.float32),
                               (((0,), (0,)), ((), ())),
                               preferred_element_type=jnp.float32)
        out_ref[0] = gsum / jnp.maximum(gcnt, 1.0)

    return pl.pallas_call(
        body,
        grid=(2,),
        in_specs=[
            pl.BlockSpec((1, _N, _HALF), lambda i: (i, 0, 0)),
            pl.BlockSpec((1, _N, _HALF), lambda i: (i, 0, 0)),
            pl.BlockSpec((2, _N, 1), lambda i: (0, 0, 0)),
            pl.BlockSpec((1, 1, _HALF), lambda i: (i, 0, 0)),
            pl.BlockSpec((1, 1, _HALF), lambda i: (i, 0, 0)),
            pl.BlockSpec((_N, 1), lambda i: (0, 0)),
        ],
        out_specs=pl.BlockSpec((1, _B, _HALF), lambda i: (i, 0, 0)),
        out_shape=jax.ShapeDtypeStruct((2, _B, _HALF), jnp.float32),
    )(h_split, s_split, cnt, g_split, be_split, gids)


def _mlp_head(qemb, pg, ne, wf_q, wf_p, wf_n, bf, gb, bb,
              w2, b2, gb2, bb2, w3, b3, gb3, bb3, w4, b4):
    def body(q_ref, pg_ref, ne_ref, wfq_ref, wfp_ref, wfn_ref, bf_ref,
             gb_ref, bb_ref, w2_ref, b2_ref, gb2_ref, bb2_ref,
             w3_ref, b3_ref, gb3_ref, bb3_ref, w4_ref, b4_ref,
             preds_ref, h3_ref):
        t1 = (jnp.dot(q_ref[...], wfq_ref[...],
                      preferred_element_type=jnp.float32) +
              jnp.dot(pg_ref[...], wfp_ref[...],
                      preferred_element_type=jnp.float32))
        rep = (lax.broadcasted_iota(jnp.int32, (_B * _K, _B), 0) // _K ==
               lax.broadcasted_iota(jnp.int32, (_B * _K, _B), 1)
               ).astype(jnp.float32)
        t1r = jnp.dot(rep, t1, preferred_element_type=jnp.float32)
        t2 = jnp.dot(ne_ref[...], wfn_ref[...],
                     preferred_element_type=jnp.float32)
        h = _bn_relu(t1r + t2 + bf_ref[...], gb_ref[...], bb_ref[...])
        h2 = _bn_relu(jnp.dot(h, w2_ref[...],
                              preferred_element_type=jnp.float32)
                      + b2_ref[...], gb2_ref[...], bb2_ref[...])
        h3 = _bn_relu(jnp.dot(h2, w3_ref[...],
                              preferred_element_type=jnp.float32)
                      + b3_ref[...], gb3_ref[...], bb3_ref[...])
        h3_ref[...] = h3
        logits = jnp.dot(h3, w4_ref[...],
                         preferred_element_type=jnp.float32) + b4_ref[...]
        preds_ref[...] = 1.0 / (1.0 + jnp.exp(-logits))

    return pl.pallas_call(
        body,
        out_shape=[
            jax.ShapeDtypeStruct((_B * _K, 1), jnp.float32),
            jax.ShapeDtypeStruct((_B * _K, _H), jnp.float32),
        ],
    )(qemb, pg, ne, wf_q, wf_p, wf_n, bf, gb, bb,
      w2, b2, gb2, bb2, w3, b3, gb3, bb3, w4, b4)


def kernel(x, edge_index, graph_ids, pgNodeEmb, neighEmb, classWeight,
           W_init, b_init, g1, be1, g2, be2, Wf, bf, gb, bb,
           Wf2, bf2, gb2, bb2, Wf3, bf3, gb3, bb3, Wf4, bf4):
    srcp, dstp, dstc = _pad_edges(edge_index)
    gids = graph_ids.reshape(_N, 1)

    w_split = W_init.reshape(20, 2, _HALF).transpose(1, 0, 2)
    b_split = b_init.reshape(2, 1, _HALF)
    g1s = g1.reshape(2, 1, _HALF)
    be1s = be1.reshape(2, 1, _HALF)
    g2s = g2.reshape(2, 1, _HALF)
    be2s = be2.reshape(2, 1, _HALF)

    h0 = _init_proj(x, w_split, b_split)
    cnt = _cnt(dstc)
    s1 = _agg(h0, srcp, dstp)
    h1 = _fold_bn_relu(h0, s1, cnt, g1s, be1s)
    s2 = _agg(h1, srcp, dstp)
    q2 = _fold_bn_relu_pool(h1, s2, cnt, g2s, be2s, gids)
    qemb = jnp.concatenate([q2[0], q2[1]], axis=1)

    preds640, h3 = _mlp_head(
        qemb, pgNodeEmb, neighEmb,
        Wf[:_H], Wf[_H:2 * _H], Wf[2 * _H:],
        bf.reshape(1, _H), gb.reshape(1, _H), bb.reshape(1, _H),
        Wf2, bf2.reshape(1, _H), gb2.reshape(1, _H), bb2.reshape(1, _H),
        Wf3, bf3.reshape(1, _H), gb3.reshape(1, _H), bb3.reshape(1, _H),
        Wf4, bf4.reshape(1, 1))
    return (preds640.reshape(_B, _K), h3)
